# Initial kernel scaffold; baseline (speedup 1.0000x reference)
import jax
import jax.numpy as jnp
from jax.experimental import pallas as pl


def kernel(inputs, edge_index, batch, edge_weight, W0, b0, gnw0, gnb0, gna0, W1, b1, gnw1, gnb1, gna1, W2, b2, gnw2, gnb2, gna2, Wdin, bdin, Wd1, bd1, Wdout, bdout):
    raise NotImplementedError("write your pallas kernel here")


# speedup vs baseline: 5.7815x; 5.7815x over previous
import functools

import jax
import jax.numpy as jnp
from jax import lax
from jax.experimental import pallas as pl
from jax.experimental.pallas import tpu as pltpu
from jax.experimental.pallas import tpu_sc as plsc

NC = 2
NS = 16
HALF = 128


def _mesh():
    return plsc.VectorSubcoreMesh(core_axis_name="c", subcore_axis_name="s",
                                  num_cores=NC, num_subcores=NS)


def _sc_deg(dst, ew, n_nodes):
    n_edges = dst.shape[0]
    per_w = n_edges // (NC * NS)
    K = 40
    n_chunks = per_w // K

    @functools.partial(
        pl.kernel,
        out_type=jax.ShapeDtypeStruct((NC, n_nodes), jnp.float32),
        mesh=_mesh(),
        scratch_types=[
            pltpu.VMEM((K,), jnp.int32),
            pltpu.VMEM((K,), jnp.float32),
            pltpu.VMEM((2000,), jnp.float32),
            pltpu.VMEM_SHARED((n_nodes,), jnp.float32),
        ],
    )
    def deg_k(dst_hbm, ew_hbm, out_hbm, idx_v, val_v, zbuf, acc_sh):
        c = lax.axis_index("c")
        s = lax.axis_index("s")

        @pl.loop(0, 125)
        def _zb(i):
            zbuf[pl.ds(i * 16, 16)] = jnp.zeros((16,), jnp.float32)

        @pl.when(s == 0)
        def _z():
            @pl.loop(0, n_nodes // 2000)
            def _(i):
                pltpu.sync_copy(zbuf, acc_sh.at[pl.ds(i * 2000, 2000)])

        plsc.subcore_barrier()
        base = (c * NS + s) * per_w

        @pl.loop(0, n_chunks)
        def _ch(k):
            off = base + k * K
            pltpu.sync_copy(dst_hbm.at[pl.ds(off, K)], idx_v)
            pltpu.sync_copy(ew_hbm.at[pl.ds(off, K)], val_v)
            pltpu.sync_copy(val_v, acc_sh.at[idx_v], add=True)

        plsc.subcore_barrier()

        @pl.when(s == 0)
        def _out():
            pltpu.sync_copy(acc_sh, out_hbm.at[c])

    return deg_k(dst, ew)


def _sc_scatter(y2, src, dst, ew, n_nodes):
    n_edges = src.shape[0]
    per_s = n_edges // NS
    K = 80
    n_chunks = per_s // K

    @functools.partial(
        pl.kernel,
        out_type=jax.ShapeDtypeStruct((NC, n_nodes, HALF), jnp.float32),
        mesh=_mesh(),
        scratch_types=[
            pltpu.VMEM((K,), jnp.int32),
            pltpu.VMEM((K,), jnp.int32),
            pltpu.VMEM((K,), jnp.float32),
            pltpu.VMEM((K, HALF), jnp.float32),
            pltpu.VMEM_SHARED((n_nodes, HALF), jnp.float32),
            pltpu.SemaphoreType.DMA,
        ],
    )
    def scat_k(y2_hbm, src_hbm, dst_hbm, ew_hbm, out_hbm,
               srcv, dstv, ewv, rows, acc_sh, sem):
        c = lax.axis_index("c")
        s = lax.axis_index("s")

        @pl.when(s == 0)
        def _init():
            pltpu.sync_copy(y2_hbm.at[pl.ds(c * n_nodes, n_nodes)], acc_sh)

        plsc.subcore_barrier()
        base = s * per_s
        coff = c * n_nodes

        @pl.loop(0, n_chunks)
        def _ch(k):
            off = base + k * K
            pltpu.sync_copy(src_hbm.at[pl.ds(off, K)], srcv)
            pltpu.sync_copy(dst_hbm.at[pl.ds(off, K)], dstv)
            pltpu.sync_copy(ew_hbm.at[pl.ds(off, K)], ewv)

            @pl.loop(0, K // 16)
            def _sh(i):
                srcv[pl.ds(i * 16, 16)] = srcv[pl.ds(i * 16, 16)] + coff

            pltpu.async_copy(y2_hbm.at[srcv], rows, sem).wait()

            @pl.loop(0, K // 16)
            def _sc(g):
                wv = ewv[pl.ds(g * 16, 16)]
                for l in range(16):
                    w = wv[l]
                    j = g * 16 + l
                    for q in range(HALF // 16):
                        rows[j, pl.ds(q * 16, 16)] = (
                            rows[j, pl.ds(q * 16, 16)] * w)

            pltpu.sync_copy(rows, acc_sh.at[dstv], add=True)

        plsc.subcore_barrier()

        @pl.when(s == 0)
        def _out():
            pltpu.sync_copy(acc_sh, out_hbm.at[c])

    return scat_k(y2, src, dst, ew)


def _mm(a, b):
    return lax.dot_general(a, b, (((1,), (0,)), ((), ())),
                           preferred_element_type=jnp.float32)


def _mmT(a, b):
    return lax.dot_general(a, b, (((0,), (0,)), ((), ())),
                           preferred_element_type=jnp.float32)


def _tc_pre(h, W, degT):
    n, _ = h.shape

    def body(h_ref, w_ref, deg_ref, y2_ref):
        deg = deg_ref[:, 0:1] + deg_ref[:, 1:2] + 1.0
        dis = lax.rsqrt(deg)
        y = _mm(h_ref[...], w_ref[...]) * dis
        y2_ref[pl.ds(0, n), :] = y[:, :HALF]
        y2_ref[pl.ds(n, n), :] = y[:, HALF:]

    return pl.pallas_call(
        body, out_shape=jax.ShapeDtypeStruct((2 * n, HALF), jnp.float32))(
            h, W, degT)


def _tc_block(msg2, degT, b, gnw, gnb, gna, batch2, h_prev):
    n = degT.shape[0]
    G = 16

    def body(*refs):
        if h_prev is not None:
            (msg_ref, deg_ref, b_ref, gw_ref, gb_ref, ga_ref,
             bt_ref, hp_ref, out_ref) = refs
        else:
            (msg_ref, deg_ref, b_ref, gw_ref, gb_ref, ga_ref,
             bt_ref, out_ref) = refs
        msg = jnp.concatenate([msg_ref[0], msg_ref[1]], axis=1)
        deg = deg_ref[:, 0:1] + deg_ref[:, 1:2] + 1.0
        dis = lax.rsqrt(deg)
        out = dis * msg + b_ref[...][None, :]

        P = (bt_ref[...] == lax.broadcasted_iota(jnp.int32, (n, G), 1)
             ).astype(jnp.float32)
        cnt = _mmT(P, jnp.ones((n, 1), jnp.float32))
        inv_cnt = 1.0 / jnp.maximum(cnt, 1.0)
        mean = _mmT(P, out) * inv_cnt
        out2 = out - ga_ref[...][None, :] * _mm(P, mean)
        var = _mmT(P, out2 * out2) * inv_cnt
        r = gw_ref[...][None, :] / jnp.sqrt(var + 1e-5)
        h = out2 * _mm(P, r) + gb_ref[...][None, :]
        h = jnp.maximum(h, 0.0)
        if h_prev is not None:
            h = h + hp_ref[...]
        out_ref[...] = h

    args = [msg2, degT, b, gnw, gnb, gna, batch2]
    aliases = {}
    if h_prev is not None:
        args.append(h_prev)
        aliases = {7: 0}
    return pl.pallas_call(
        body, out_shape=jax.ShapeDtypeStruct((n, 256), jnp.float32),
        input_output_aliases=aliases)(*args)


def _tc_head(h, batch2, Wdin, bdin, Wd1, bd1, Wdout, bdout):
    n = h.shape[0]
    G = 16

    def body(h_ref, bt_ref, w0_ref, b0_ref, w1_ref, b1_ref, w2_ref, b2_ref,
             out_ref):
        P = (bt_ref[...] == lax.broadcasted_iota(jnp.int32, (n, G), 1)
             ).astype(jnp.float32)
        cnt = _mmT(P, jnp.ones((n, 1), jnp.float32))
        inv_cnt = 1.0 / jnp.maximum(cnt, 1.0)
        flat = _mmT(P, h_ref[...]) * inv_cnt
        z = jnp.maximum(_mm(flat, w0_ref[...]) + b0_ref[...][None, :], 0.0)
        z = jnp.maximum(_mm(z, w1_ref[...]) + b1_ref[...][None, :], 0.0)
        out_ref[...] = _mm(z, w2_ref[...]) + b2_ref[...][None, :]

    return pl.pallas_call(
        body, out_shape=jax.ShapeDtypeStruct((G, Wdout.shape[1]), jnp.float32))(
            h, batch2, Wdin, bdin, Wd1, bd1, Wdout, bdout)


def kernel(inputs, edge_index, batch, edge_weight, W0, b0, gnw0, gnb0, gna0,
           W1, b1, gnw1, gnb1, gna1, W2, b2, gnw2, gnb2, gna2,
           Wdin, bdin, Wd1, bd1, Wdout, bdout):
    n = inputs.shape[0]
    src = edge_index[0]
    dst = edge_index[1]

    deg2 = _sc_deg(dst, edge_weight, n)
    degT = deg2.T
    batch2 = batch.reshape(-1, 1)

    h = inputs
    for (W, b, gnw, gnb, gna, res) in (
            (W0, b0, gnw0, gnb0, gna0, False),
            (W1, b1, gnw1, gnb1, gna1, True),
            (W2, b2, gnw2, gnb2, gna2, True)):
        y2 = _tc_pre(h, W, degT)
        msg2 = _sc_scatter(y2, src, dst, edge_weight, n)
        h = _tc_block(msg2, degT, b, gnw, gnb, gna, batch2,
                      h if res else None)

    return _tc_head(h, batch2, Wdin, bdin, Wd1, bd1, Wdout, bdout)


# speedup vs baseline: 13.6989x; 2.3694x over previous
import functools

import jax
import jax.numpy as jnp
from jax import lax
from jax.experimental import pallas as pl
from jax.experimental.pallas import tpu as pltpu
from jax.experimental.pallas import tpu_sc as plsc

NC = 2
NS = 16
HALF = 128


def _mesh():
    return plsc.VectorSubcoreMesh(core_axis_name="c", subcore_axis_name="s",
                                  num_cores=NC, num_subcores=NS)


def _sc_deg(dst3, ew3, n_nodes):
    _, n_chunks, K = dst3.shape

    @functools.partial(
        pl.kernel,
        out_type=jax.ShapeDtypeStruct((NC, n_nodes), jnp.float32),
        mesh=_mesh(),
        scratch_types=[
            pltpu.VMEM((n_chunks, K), jnp.int32),
            pltpu.VMEM((n_chunks, K), jnp.float32),
            pltpu.VMEM((2000,), jnp.float32),
            pltpu.VMEM_SHARED((n_nodes,), jnp.float32),
        ],
    )
    def deg_k(dst_hbm, ew_hbm, out_hbm, idx_v, val_v, zbuf, acc_sh):
        c = lax.axis_index("c")
        s = lax.axis_index("s")
        w = c * NS + s
        pltpu.sync_copy(dst_hbm.at[w], idx_v)
        pltpu.sync_copy(ew_hbm.at[w], val_v)

        @pl.loop(0, 125)
        def _zb(i):
            zbuf[pl.ds(i * 16, 16)] = jnp.zeros((16,), jnp.float32)

        @pl.when(s == 0)
        def _z():
            @pl.loop(0, n_nodes // 2000)
            def _(i):
                pltpu.sync_copy(zbuf, acc_sh.at[pl.ds(i * 2000, 2000)])

        plsc.subcore_barrier()

        @pl.loop(0, n_chunks)
        def _ch(k):
            pltpu.sync_copy(val_v.at[k], acc_sh.at[idx_v.at[k]], add=True)

        plsc.subcore_barrier()

        @pl.when(s == 0)
        def _out():
            pltpu.sync_copy(acc_sh, out_hbm.at[c])

    return deg_k(dst3, ew3)


def _sc_scatter(y2, src3, dst3, ew3, n_nodes):
    _, n_chunks, K = src3.shape

    @functools.partial(
        pl.kernel,
        out_type=jax.ShapeDtypeStruct((NC, n_nodes, HALF), jnp.float32),
        mesh=_mesh(),
        scratch_types=[
            pltpu.VMEM((n_chunks, K), jnp.int32),
            pltpu.VMEM((K,), jnp.int32),
            pltpu.VMEM((K,), jnp.int32),
            pltpu.VMEM((K,), jnp.float32),
            pltpu.VMEM((K,), jnp.float32),
            pltpu.VMEM((K, HALF), jnp.float32),
            pltpu.VMEM((K, HALF), jnp.float32),
            pltpu.VMEM_SHARED((n_nodes, HALF), jnp.float32),
            pltpu.SemaphoreType.DMA,
            pltpu.SemaphoreType.DMA,
            pltpu.SemaphoreType.DMA,
            pltpu.SemaphoreType.DMA,
            pltpu.SemaphoreType.DMA,
            pltpu.SemaphoreType.DMA,
        ],
    )
    def scat_k(y2_hbm, src_hbm, dst_hbm, ew_hbm, out_hbm,
               srcb, dstv0, dstv1, ewv0, ewv1, rows0, rows1, acc_sh,
               sem0, sem1, esem0, esem1, dsem0, dsem1):
        c = lax.axis_index("c")
        s = lax.axis_index("s")
        coff = c * n_nodes

        pltpu.sync_copy(src_hbm.at[s], srcb)

        @pl.when(s == 0)
        def _init():
            pltpu.sync_copy(y2_hbm.at[pl.ds(c * n_nodes, n_nodes)], acc_sh)

        @pl.loop(0, n_chunks)
        def _sh(r):
            for g in range(K // 16):
                srcb[r, pl.ds(g * 16, 16)] = srcb[r, pl.ds(g * 16, 16)] + coff

        plsc.subcore_barrier()

        def start_chunk(k, rows, sem, ewv, esem, dstv, dsem):
            pltpu.async_copy(y2_hbm.at[srcb.at[k]], rows, sem)
            pltpu.async_copy(ew_hbm.at[s, k], ewv, esem)
            pltpu.async_copy(dst_hbm.at[s, k], dstv, dsem)

        def wait_chunk(rows, sem, ewv, esem, dstv, dsem):
            pltpu.make_async_copy(y2_hbm.at[pl.ds(0, K)], rows, sem).wait()
            pltpu.make_async_copy(ew_hbm.at[0, 0], ewv, esem).wait()
            pltpu.make_async_copy(dst_hbm.at[0, 0], dstv, dsem).wait()

        def scale_scatter(rows, ewv, dstv):
            @pl.loop(0, K // 16)
            def _sc(g):
                wv = ewv[pl.ds(g * 16, 16)]
                for l in range(16):
                    w = wv[l]
                    j = g * 16 + l
                    for q in range(HALF // 16):
                        rows[j, pl.ds(q * 16, 16)] = (
                            rows[j, pl.ds(q * 16, 16)] * w)

            pltpu.sync_copy(rows, acc_sh.at[dstv], add=True)

        start_chunk(0, rows0, sem0, ewv0, esem0, dstv0, dsem0)

        @pl.loop(0, (n_chunks - 1) // 2)
        def _pipe(i):
            k0 = 2 * i
            wait_chunk(rows0, sem0, ewv0, esem0, dstv0, dsem0)
            start_chunk(k0 + 1, rows1, sem1, ewv1, esem1, dstv1, dsem1)
            scale_scatter(rows0, ewv0, dstv0)
            wait_chunk(rows1, sem1, ewv1, esem1, dstv1, dsem1)
            start_chunk(k0 + 2, rows0, sem0, ewv0, esem0, dstv0, dsem0)
            scale_scatter(rows1, ewv1, dstv1)

        wait_chunk(rows0, sem0, ewv0, esem0, dstv0, dsem0)
        scale_scatter(rows0, ewv0, dstv0)

        plsc.subcore_barrier()

        @pl.when(s == 0)
        def _out():
            pltpu.sync_copy(acc_sh, out_hbm.at[c])

    return scat_k(y2, src3, dst3, ew3)


def _mm(a, b):
    return lax.dot_general(a, b, (((1,), (0,)), ((), ())),
                           preferred_element_type=jnp.float32)


def _mmT(a, b):
    return lax.dot_general(a, b, (((0,), (0,)), ((), ())),
                           preferred_element_type=jnp.float32)


def _tc_pre(h, W, degT):
    n, _ = h.shape

    def body(h_ref, w_ref, deg_ref, y2_ref):
        deg = deg_ref[:, 0:1] + deg_ref[:, 1:2] + 1.0
        dis = lax.rsqrt(deg)
        y = _mm(h_ref[...], w_ref[...]) * dis
        y2_ref[pl.ds(0, n), :] = y[:, :HALF]
        y2_ref[pl.ds(n, n), :] = y[:, HALF:]

    return pl.pallas_call(
        body, out_shape=jax.ShapeDtypeStruct((2 * n, HALF), jnp.float32))(
            h, W, degT)


def _tc_block(msg2, degT, b, gnw, gnb, gna, batch2, h_prev):
    n = degT.shape[0]
    G = 16

    def body(*refs):
        if h_prev is not None:
            (msg_ref, deg_ref, b_ref, gw_ref, gb_ref, ga_ref,
             bt_ref, hp_ref, out_ref) = refs
        else:
            (msg_ref, deg_ref, b_ref, gw_ref, gb_ref, ga_ref,
             bt_ref, out_ref) = refs
        msg = jnp.concatenate([msg_ref[0], msg_ref[1]], axis=1)
        deg = deg_ref[:, 0:1] + deg_ref[:, 1:2] + 1.0
        dis = lax.rsqrt(deg)
        out = dis * msg + b_ref[...][None, :]

        P = (bt_ref[...] == lax.broadcasted_iota(jnp.int32, (n, G), 1)
             ).astype(jnp.float32)
        cnt = _mmT(P, jnp.ones((n, 1), jnp.float32))
        inv_cnt = 1.0 / jnp.maximum(cnt, 1.0)
        mean = _mmT(P, out) * inv_cnt
        out2 = out - ga_ref[...][None, :] * _mm(P, mean)
        var = _mmT(P, out2 * out2) * inv_cnt
        r = gw_ref[...][None, :] / jnp.sqrt(var + 1e-5)
        h = out2 * _mm(P, r) + gb_ref[...][None, :]
        h = jnp.maximum(h, 0.0)
        if h_prev is not None:
            h = h + hp_ref[...]
        out_ref[...] = h

    args = [msg2, degT, b, gnw, gnb, gna, batch2]
    aliases = {}
    if h_prev is not None:
        args.append(h_prev)
        aliases = {7: 0}
    return pl.pallas_call(
        body, out_shape=jax.ShapeDtypeStruct((n, 256), jnp.float32),
        input_output_aliases=aliases)(*args)


def _tc_head(h, batch2, Wdin, bdin, Wd1, bd1, Wdout, bdout):
    n = h.shape[0]
    G = 16

    def body(h_ref, bt_ref, w0_ref, b0_ref, w1_ref, b1_ref, w2_ref, b2_ref,
             out_ref):
        P = (bt_ref[...] == lax.broadcasted_iota(jnp.int32, (n, G), 1)
             ).astype(jnp.float32)
        cnt = _mmT(P, jnp.ones((n, 1), jnp.float32))
        inv_cnt = 1.0 / jnp.maximum(cnt, 1.0)
        flat = _mmT(P, h_ref[...]) * inv_cnt
        z = jnp.maximum(_mm(flat, w0_ref[...]) + b0_ref[...][None, :], 0.0)
        z = jnp.maximum(_mm(z, w1_ref[...]) + b1_ref[...][None, :], 0.0)
        out_ref[...] = _mm(z, w2_ref[...]) + b2_ref[...][None, :]

    return pl.pallas_call(
        body, out_shape=jax.ShapeDtypeStruct((G, Wdout.shape[1]), jnp.float32))(
            h, batch2, Wdin, bdin, Wd1, bd1, Wdout, bdout)


def kernel(inputs, edge_index, batch, edge_weight, W0, b0, gnw0, gnb0, gna0,
           W1, b1, gnw1, gnb1, gna1, W2, b2, gnw2, gnb2, gna2,
           Wdin, bdin, Wd1, bd1, Wdout, bdout):
    n = inputs.shape[0]
    src = edge_index[0]
    dst = edge_index[1]

    n_edges = src.shape[0]
    dstd = dst.reshape(NC * NS, n_edges // (NC * NS * 125), 125)
    ewd = edge_weight.reshape(dstd.shape)
    src3 = src.reshape(NS, n_edges // (NS * 80), 80)
    dst3 = dst.reshape(src3.shape)
    ew3 = edge_weight.reshape(src3.shape)

    deg2 = _sc_deg(dstd, ewd, n)
    degT = deg2.T
    batch2 = batch.reshape(-1, 1)

    h = inputs
    for (W, b, gnw, gnb, gna, res) in (
            (W0, b0, gnw0, gnb0, gna0, False),
            (W1, b1, gnw1, gnb1, gna1, True),
            (W2, b2, gnw2, gnb2, gna2, True)):
        y2 = _tc_pre(h, W, degT)
        msg2 = _sc_scatter(y2, src3, dst3, ew3, n)
        h = _tc_block(msg2, degT, b, gnw, gnb, gna, batch2,
                      h if res else None)

    return _tc_head(h, batch2, Wdin, bdin, Wd1, bd1, Wdout, bdout)
